# baseline (speedup 1.0000x reference)
import functools

import jax
import jax.numpy as jnp
from jax import lax
from jax.experimental import pallas as pl
from jax.experimental.pallas import tpu as pltpu
from jax.experimental.pallas import tpu_sc as plsc

_BATCH = 16384
_N_FIELDS = 26
_DIM = 64

_NC = 2
_NS = 16
_NW = _NC * _NS

_B_PER_W = _BATCH // _NW
_CB = 16
_ROWS = _CB * _N_FIELDS
_NBUF = 4
_N_CHUNKS = _B_PER_W // _CB
_N_GROUPS = _N_CHUNKS // _NBUF
assert _N_GROUPS * _NBUF == _N_CHUNKS


def _gather_body(idx_hbm, table_hbm, out_hbm, idx_v, rows_v, gsems, wsems):
    wid = lax.axis_index("s") * _NC + lax.axis_index("c")
    base_b = wid * _B_PER_W

    def start_gather(b, chunk_i):
        bb = base_b + chunk_i * _CB
        pltpu.sync_copy(idx_hbm.at[pl.ds(bb, _CB)], idx_v.at[b])
        for j in range(_CB):
            pltpu.async_copy(
                table_hbm.at[idx_v.at[b, j]],
                rows_v.at[b, j],
                gsems.at[b],
            )

    def wait_gather(b):
        for j in range(_CB):
            pltpu.make_async_copy(
                table_hbm.at[idx_v.at[b, j]],
                rows_v.at[b, j],
                gsems.at[b],
            ).wait()

    def start_writeback(b, chunk_i):
        bb = base_b + chunk_i * _CB
        pltpu.async_copy(rows_v.at[b], out_hbm.at[pl.ds(bb, _CB)], wsems.at[b])

    def wait_writeback(b):
        pltpu.make_async_copy(
            rows_v.at[b], out_hbm.at[pl.ds(base_b, _CB)], wsems.at[b]
        ).wait()

    for b in range(_NBUF):
        start_gather(b, b)

    def group(j, carry):
        for b in range(_NBUF):
            i = j * _NBUF + b
            wait_gather(b)
            start_writeback(b, i)

            @pl.when(j < _N_GROUPS - 1)
            def _():
                wait_writeback(b)
                start_gather(b, i + _NBUF)

        return carry

    lax.fori_loop(0, _N_GROUPS, group, 0)

    for b in range(_NBUF):
        wait_writeback(b)


@functools.partial(
    pl.kernel,
    mesh=plsc.VectorSubcoreMesh(core_axis_name="c", subcore_axis_name="s"),
    out_type=jax.ShapeDtypeStruct((_BATCH, _N_FIELDS, _DIM), jnp.float32),
    scratch_types=[
        pltpu.VMEM((_NBUF, _CB, _N_FIELDS), jnp.int32),
        pltpu.VMEM((_NBUF, _CB, _N_FIELDS, _DIM), jnp.float32),
        pltpu.SemaphoreType.DMA((_NBUF,)),
        pltpu.SemaphoreType.DMA((_NBUF,)),
    ],
    compiler_params=pltpu.CompilerParams(use_tc_tiling_on_sc=False),
)
def _gather_call(idx_hbm, table_hbm, out_hbm, idx_v, rows_v, gsems, wsems):
    _gather_body(idx_hbm, table_hbm, out_hbm, idx_v, rows_v, gsems, wsems)


@jax.jit
def kernel(X, embeddings):
    return _gather_call(X.astype(jnp.int32), embeddings)


# speedup vs baseline: 1.0151x; 1.0151x over previous
import functools

import jax
import jax.numpy as jnp
from jax import lax
from jax.experimental import pallas as pl
from jax.experimental.pallas import tpu as pltpu
from jax.experimental.pallas import tpu_sc as plsc

_BATCH = 16384
_N_FIELDS = 26
_DIM = 64

_NC = 2
_NS = 16
_NW = _NC * _NS

_B_PER_W = _BATCH // _NW
_NBUF = 2
_N_PAIRS = _N_FIELDS // _NBUF


def _gather_body(idxT_hbm, table_hbm, out_hbm, idx_v, rows_v, gsems, wsems):
    wid = lax.axis_index("s") * _NC + lax.axis_index("c")
    b0 = wid * _B_PER_W

    def start_gather(b, f):
        pltpu.async_copy(
            table_hbm.at[idx_v.at[f]], rows_v.at[b], gsems.at[b]
        )

    def wait_gather(b, f):
        pltpu.make_async_copy(
            table_hbm.at[idx_v.at[f]], rows_v.at[b], gsems.at[b]
        ).wait()

    def start_writeback(b, f):
        pltpu.async_copy(
            rows_v.at[b], out_hbm.at[pl.ds(b0, _B_PER_W), f], wsems.at[b]
        )

    def wait_writeback(b):
        pltpu.make_async_copy(
            rows_v.at[b], out_hbm.at[pl.ds(b0, _B_PER_W), 0], wsems.at[b]
        ).wait()

    pltpu.sync_copy(idxT_hbm.at[:, pl.ds(b0, _B_PER_W)], idx_v)

    for b in range(_NBUF):
        start_gather(b, b)

    def pair(j, carry):
        for b in range(_NBUF):
            f = j * _NBUF + b
            wait_gather(b, f)
            start_writeback(b, f)

            @pl.when(j < _N_PAIRS - 1)
            def _():
                wait_writeback(b)
                start_gather(b, f + _NBUF)

        return carry

    lax.fori_loop(0, _N_PAIRS, pair, 0)

    for b in range(_NBUF):
        wait_writeback(b)


@functools.partial(
    pl.kernel,
    mesh=plsc.VectorSubcoreMesh(core_axis_name="c", subcore_axis_name="s"),
    out_type=jax.ShapeDtypeStruct((_BATCH, _N_FIELDS, _DIM), jnp.float32),
    scratch_types=[
        pltpu.VMEM((_N_FIELDS, _B_PER_W), jnp.int32),
        pltpu.VMEM((_NBUF, _B_PER_W, _DIM), jnp.float32),
        pltpu.SemaphoreType.DMA((_NBUF,)),
        pltpu.SemaphoreType.DMA((_NBUF,)),
    ],
    compiler_params=pltpu.CompilerParams(use_tc_tiling_on_sc=False),
)
def _gather_call(idxT_hbm, table_hbm, out_hbm, idx_v, rows_v, gsems, wsems):
    _gather_body(idxT_hbm, table_hbm, out_hbm, idx_v, rows_v, gsems, wsems)


@jax.jit
def kernel(X, embeddings):
    return _gather_call(X.T.astype(jnp.int32), embeddings)
